# baseline (speedup 1.0000x reference)
import functools

import jax
import jax.numpy as jnp
from jax import lax
from jax.experimental import pallas as pl
from jax.experimental.pallas import tpu as pltpu
from jax.experimental.pallas import tpu_sc as plsc

N = 10000
NPAD = 10240
E = 320000
D = 128
HID = 256
NSC = 2
NTILE = 16
NW = NSC * NTILE
KC = 128
NCK = 80
EPT = NCK * KC
EPAD = NW * EPT
RPT = NPAD // NTILE
NBUF = 4

_mesh = plsc.VectorSubcoreMesh(core_axis_name="c", subcore_axis_name="s")


@functools.partial(
    pl.kernel,
    out_type=jax.ShapeDtypeStruct((NSC, NPAD, D), jnp.float32),
    mesh=_mesh,
    scratch_types=[
        pltpu.VMEM((NCK, KC), jnp.int32),
        pltpu.VMEM((KC, D), jnp.float32),
        pltpu.VMEM_SHARED((NPAD, D), jnp.float32),
        pltpu.SemaphoreType.DMA,
        pltpu.SemaphoreType.DMA,
    ],
)
def _degree(dst_hbm, zeros_hbm, ones_hbm, hist_hbm,
            dsts, ones_v, acc_sh, s0, s1):
    ssem = (s0, s1)
    c = lax.axis_index("c")
    s = lax.axis_index("s")
    wid = c * NTILE + s
    pltpu.sync_copy(zeros_hbm.at[pl.ds(s * RPT, RPT)],
                    acc_sh.at[pl.ds(s * RPT, RPT)])
    pltpu.sync_copy(dst_hbm.at[wid], dsts)
    pltpu.sync_copy(ones_hbm, ones_v)
    plsc.subcore_barrier()

    def body(g, carry):
        for b in range(2):
            j = g * 2 + b
            @pl.when(g >= 1)
            def _():
                pltpu.make_async_copy(
                    ones_v, acc_sh.at[dsts.at[j - 2]], ssem[b]).wait()
            pltpu.async_copy(ones_v, acc_sh.at[dsts.at[j]], ssem[b], add=True)
        return carry

    lax.fori_loop(0, NCK // 2, body, 0)
    for b in range(2):
        pltpu.make_async_copy(
            ones_v, acc_sh.at[dsts.at[NCK - 2 + b]], ssem[b]).wait()
    plsc.subcore_barrier()
    pltpu.sync_copy(acc_sh.at[pl.ds(s * RPT, RPT)],
                    hist_hbm.at[c, pl.ds(s * RPT, RPT)])


@functools.partial(
    pl.kernel,
    out_type=jax.ShapeDtypeStruct((NSC, NPAD, D), jnp.float32),
    mesh=_mesh,
    scratch_types=[
        pltpu.VMEM((NCK, KC), jnp.int32),
        pltpu.VMEM((2, KC), jnp.int32),
        pltpu.VMEM((KC, D), jnp.float32),
        pltpu.VMEM((KC, D), jnp.float32),
        pltpu.VMEM_SHARED((NPAD, D), jnp.float32),
        pltpu.SemaphoreType.DMA,
        pltpu.SemaphoreType.DMA,
        pltpu.SemaphoreType.DMA,
        pltpu.SemaphoreType.DMA,
        pltpu.SemaphoreType.DMA,
        pltpu.SemaphoreType.DMA,
    ],
)
def _spmm(g_hbm, src_hbm, dst_hbm, zeros_hbm, acc_hbm,
          srcs, dst_t, r0, r1, acc_sh,
          g0, g1, s0, s1, d0, d1):
    rows = (r0, r1)
    gsem = (g0, g1)
    ssem = (s0, s1)
    dsem = (d0, d1)
    c = lax.axis_index("c")
    s = lax.axis_index("s")
    wid = c * NTILE + s
    pltpu.sync_copy(zeros_hbm.at[pl.ds(s * RPT, RPT)],
                    acc_sh.at[pl.ds(s * RPT, RPT)])
    pltpu.sync_copy(src_hbm.at[wid], srcs)
    plsc.subcore_barrier()

    def body(g, carry):
        for b in range(2):
            j = g * 2 + b

            pltpu.async_copy(dst_hbm.at[wid, j], dst_t.at[b], dsem[b])
            pltpu.async_copy(g_hbm.at[srcs.at[j]], rows[b], gsem[b])
            pltpu.make_async_copy(
                g_hbm.at[srcs.at[j]], rows[b], gsem[b]).wait()
            pltpu.make_async_copy(
                dst_hbm.at[wid, j], dst_t.at[b], dsem[b]).wait()
            pltpu.sync_copy(rows[b], acc_sh.at[dst_t.at[b]], add=True)
        return carry

    lax.fori_loop(0, NCK // 2, body, 0)
    plsc.subcore_barrier()
    pltpu.sync_copy(acc_sh.at[pl.ds(s * RPT, RPT)],
                    acc_hbm.at[c, pl.ds(s * RPT, RPT)])


def _dis_g1_body(hist_ref, x_ref, dis_ref, g1_ref):
    deg = hist_ref[0, :, 0:1] + hist_ref[1, :, 0:1] + 1.0
    dis = lax.rsqrt(deg)
    dis_ref[...] = dis
    g1_ref[...] = x_ref[...] * dis


def _mid_body(acc_ref, dis_ref, g1_ref, w1_ref, b1_ref, w2_ref, g2_ref):
    dis = dis_ref[...]
    s1 = dis * (acc_ref[0] + acc_ref[1] + g1_ref[...])
    h = jnp.maximum(
        jnp.dot(s1, w1_ref[...], preferred_element_type=jnp.float32)
        + b1_ref[...], 0.0)
    t = jnp.dot(h, w2_ref[...], preferred_element_type=jnp.float32)
    g2_ref[...] = dis * t


def _final_body(acc_ref, dis_ref, g2_ref, b2_ref, out_ref):
    dis = dis_ref[...]
    out_ref[...] = dis * (acc_ref[0] + acc_ref[1] + g2_ref[...]) + b2_ref[...]


def kernel(x, edge_index, W1, b1, W2, b2):
    src = edge_index[0].astype(jnp.int32)
    dst = edge_index[1].astype(jnp.int32)
    src_p = jnp.concatenate(
        [src, jnp.zeros((EPAD - E,), jnp.int32)]).reshape(NW, NCK, KC)
    dst_p = jnp.concatenate(
        [dst, N + jnp.arange(EPAD - E, dtype=jnp.int32) % (NPAD - N)]
    ).reshape(NW, NCK, KC)
    x_pad = jnp.zeros((NPAD, D), jnp.float32).at[:N].set(x)
    zeros_nd = jnp.zeros((NPAD, D), jnp.float32)
    ones_kd = jnp.ones((KC, D), jnp.float32)

    hist = _degree(dst_p, zeros_nd, ones_kd)

    rb = 1024
    dis, g1 = pl.pallas_call(
        _dis_g1_body,
        grid=(NPAD // rb,),
        in_specs=[
            pl.BlockSpec((NSC, rb, D), lambda i: (0, i, 0)),
            pl.BlockSpec((rb, D), lambda i: (i, 0)),
        ],
        out_specs=[
            pl.BlockSpec((rb, 1), lambda i: (i, 0)),
            pl.BlockSpec((rb, D), lambda i: (i, 0)),
        ],
        out_shape=[
            jax.ShapeDtypeStruct((NPAD, 1), jnp.float32),
            jax.ShapeDtypeStruct((NPAD, D), jnp.float32),
        ],
    )(hist, x_pad)

    acc1 = _spmm(g1, src_p, dst_p, zeros_nd)

    g2 = pl.pallas_call(
        _mid_body,
        grid=(NPAD // rb,),
        in_specs=[
            pl.BlockSpec((NSC, rb, D), lambda i: (0, i, 0)),
            pl.BlockSpec((rb, 1), lambda i: (i, 0)),
            pl.BlockSpec((rb, D), lambda i: (i, 0)),
            pl.BlockSpec((D, HID), lambda i: (0, 0)),
            pl.BlockSpec((1, HID), lambda i: (0, 0)),
            pl.BlockSpec((HID, D), lambda i: (0, 0)),
        ],
        out_specs=pl.BlockSpec((rb, D), lambda i: (i, 0)),
        out_shape=jax.ShapeDtypeStruct((NPAD, D), jnp.float32),
    )(acc1, dis, g1, W1, b1.reshape(1, HID), W2)

    acc2 = _spmm(g2, src_p, dst_p, zeros_nd)

    rf = 2000
    out = pl.pallas_call(
        _final_body,
        grid=(N // rf,),
        in_specs=[
            pl.BlockSpec((NSC, rf, D), lambda i: (0, i, 0)),
            pl.BlockSpec((rf, 1), lambda i: (i, 0)),
            pl.BlockSpec((rf, D), lambda i: (i, 0)),
            pl.BlockSpec((1, D), lambda i: (0, 0)),
        ],
        out_specs=pl.BlockSpec((rf, D), lambda i: (i, 0)),
        out_shape=jax.ShapeDtypeStruct((N, D), jnp.float32),
    )(acc2, dis, g2, b2.reshape(1, D))
    return out


# speedup vs baseline: 2.8148x; 2.8148x over previous
import functools

import jax
import jax.numpy as jnp
from jax import lax
from jax.experimental import pallas as pl
from jax.experimental.pallas import tpu as pltpu
from jax.experimental.pallas import tpu_sc as plsc

N = 10000
NPAD = 10240
E = 320000
D = 128
HID = 256
NSC = 2
NTILE = 16
NW = NSC * NTILE
KC = 128
NCK = 80
EPT = NCK * KC
EPAD = NW * EPT
RPT = NPAD // NTILE
NBUF = 4

_mesh = plsc.VectorSubcoreMesh(core_axis_name="c", subcore_axis_name="s")


@functools.partial(
    pl.kernel,
    out_type=jax.ShapeDtypeStruct((NSC, NPAD, D), jnp.float32),
    mesh=_mesh,
    scratch_types=[
        pltpu.VMEM((NCK, KC), jnp.int32),
        pltpu.VMEM((KC, D), jnp.float32),
        pltpu.VMEM_SHARED((NPAD, D), jnp.float32),
        pltpu.SemaphoreType.DMA,
        pltpu.SemaphoreType.DMA,
    ],
)
def _degree(dst_hbm, zeros_hbm, ones_hbm, hist_hbm,
            dsts, ones_v, acc_sh, s0, s1):
    ssem = (s0, s1)
    c = lax.axis_index("c")
    s = lax.axis_index("s")
    wid = c * NTILE + s
    pltpu.sync_copy(zeros_hbm.at[pl.ds(s * RPT, RPT)],
                    acc_sh.at[pl.ds(s * RPT, RPT)])
    pltpu.sync_copy(dst_hbm.at[wid], dsts)
    pltpu.sync_copy(ones_hbm, ones_v)
    plsc.subcore_barrier()

    def body(g, carry):
        for b in range(2):
            j = g * 2 + b
            @pl.when(g >= 1)
            def _():
                pltpu.make_async_copy(
                    ones_v, acc_sh.at[dsts.at[j - 2]], ssem[b]).wait()
            pltpu.async_copy(ones_v, acc_sh.at[dsts.at[j]], ssem[b], add=True)
        return carry

    lax.fori_loop(0, NCK // 2, body, 0)
    for b in range(2):
        pltpu.make_async_copy(
            ones_v, acc_sh.at[dsts.at[NCK - 2 + b]], ssem[b]).wait()
    plsc.subcore_barrier()
    pltpu.sync_copy(acc_sh.at[pl.ds(s * RPT, RPT)],
                    hist_hbm.at[c, pl.ds(s * RPT, RPT)])


@functools.partial(
    pl.kernel,
    out_type=jax.ShapeDtypeStruct((NSC, NPAD, D), jnp.float32),
    mesh=_mesh,
    scratch_types=[
        pltpu.VMEM((NCK, KC), jnp.int32),
        pltpu.VMEM((2, KC), jnp.int32),
        pltpu.VMEM((KC, D), jnp.float32),
        pltpu.VMEM((KC, D), jnp.float32),
        pltpu.VMEM_SHARED((NPAD, D), jnp.float32),
        pltpu.SemaphoreType.DMA,
        pltpu.SemaphoreType.DMA,
        pltpu.SemaphoreType.DMA,
        pltpu.SemaphoreType.DMA,
        pltpu.SemaphoreType.DMA,
        pltpu.SemaphoreType.DMA,
    ],
)
def _spmm(g_hbm, src_hbm, dst_hbm, zeros_hbm, acc_hbm,
          srcs, dst_t, r0, r1, acc_sh,
          g0, g1, s0, s1, d0, d1):
    rows = (r0, r1)
    gsem = (g0, g1)
    ssem = (s0, s1)
    dsem = (d0, d1)
    c = lax.axis_index("c")
    s = lax.axis_index("s")
    wid = c * NTILE + s
    pltpu.sync_copy(zeros_hbm.at[pl.ds(s * RPT, RPT)],
                    acc_sh.at[pl.ds(s * RPT, RPT)])
    pltpu.sync_copy(src_hbm.at[wid], srcs)
    plsc.subcore_barrier()

    def body(g, carry):
        for b in range(2):
            j = g * 2 + b

            @pl.when(j >= 2)
            def _():
                pltpu.make_async_copy(
                    rows[b], acc_sh.at[dst_t.at[b]], ssem[b]).wait()
            pltpu.async_copy(dst_hbm.at[wid, j], dst_t.at[b], dsem[b])
            pltpu.async_copy(g_hbm.at[srcs.at[j]], rows[b], gsem[b])
            pltpu.make_async_copy(
                g_hbm.at[srcs.at[j]], rows[b], gsem[b]).wait()
            pltpu.make_async_copy(
                dst_hbm.at[wid, j], dst_t.at[b], dsem[b]).wait()
            pltpu.async_copy(rows[b], acc_sh.at[dst_t.at[b]], ssem[b],
                             add=True)
        return carry

    lax.fori_loop(0, NCK // 2, body, 0)
    for b in range(2):
        pltpu.make_async_copy(
            rows[b], acc_sh.at[dst_t.at[b]], ssem[b]).wait()
    plsc.subcore_barrier()
    pltpu.sync_copy(acc_sh.at[pl.ds(s * RPT, RPT)],
                    acc_hbm.at[c, pl.ds(s * RPT, RPT)])


def _dis_g1_body(hist_ref, x_ref, dis_ref, g1_ref):
    deg = hist_ref[0, :, 0:1] + hist_ref[1, :, 0:1] + 1.0
    dis = lax.rsqrt(deg)
    dis_ref[...] = dis
    g1_ref[...] = x_ref[...] * dis


def _mid_body(acc_ref, dis_ref, g1_ref, w1_ref, b1_ref, w2_ref, g2_ref):
    dis = dis_ref[...]
    s1 = dis * (acc_ref[0] + acc_ref[1] + g1_ref[...])
    h = jnp.maximum(
        jnp.dot(s1, w1_ref[...], preferred_element_type=jnp.float32)
        + b1_ref[...], 0.0)
    t = jnp.dot(h, w2_ref[...], preferred_element_type=jnp.float32)
    g2_ref[...] = dis * t


def _final_body(acc_ref, dis_ref, g2_ref, b2_ref, out_ref):
    dis = dis_ref[...]
    out_ref[...] = dis * (acc_ref[0] + acc_ref[1] + g2_ref[...]) + b2_ref[...]


def kernel(x, edge_index, W1, b1, W2, b2):
    src = edge_index[0].astype(jnp.int32)
    dst = edge_index[1].astype(jnp.int32)
    npad_w = (EPAD - E) // NW
    nreal_w = E // NW
    pad_src = (jnp.arange(NW * npad_w, dtype=jnp.int32) * 37) % N
    pad_dst = N + jnp.arange(NW * npad_w, dtype=jnp.int32) % (NPAD - N)
    src_p = jnp.concatenate(
        [src.reshape(NW, nreal_w), pad_src.reshape(NW, npad_w)],
        axis=1).reshape(NW, NCK, KC)
    dst_p = jnp.concatenate(
        [dst.reshape(NW, nreal_w), pad_dst.reshape(NW, npad_w)],
        axis=1).reshape(NW, NCK, KC)
    x_pad = jnp.zeros((NPAD, D), jnp.float32).at[:N].set(x)
    zeros_nd = jnp.zeros((NPAD, D), jnp.float32)
    ones_kd = jnp.ones((KC, D), jnp.float32)

    hist = _degree(dst_p, zeros_nd, ones_kd)

    rb = 1024
    dis, g1 = pl.pallas_call(
        _dis_g1_body,
        grid=(NPAD // rb,),
        in_specs=[
            pl.BlockSpec((NSC, rb, D), lambda i: (0, i, 0)),
            pl.BlockSpec((rb, D), lambda i: (i, 0)),
        ],
        out_specs=[
            pl.BlockSpec((rb, 1), lambda i: (i, 0)),
            pl.BlockSpec((rb, D), lambda i: (i, 0)),
        ],
        out_shape=[
            jax.ShapeDtypeStruct((NPAD, 1), jnp.float32),
            jax.ShapeDtypeStruct((NPAD, D), jnp.float32),
        ],
    )(hist, x_pad)

    acc1 = _spmm(g1, src_p, dst_p, zeros_nd)

    g2 = pl.pallas_call(
        _mid_body,
        grid=(NPAD // rb,),
        in_specs=[
            pl.BlockSpec((NSC, rb, D), lambda i: (0, i, 0)),
            pl.BlockSpec((rb, 1), lambda i: (i, 0)),
            pl.BlockSpec((rb, D), lambda i: (i, 0)),
            pl.BlockSpec((D, HID), lambda i: (0, 0)),
            pl.BlockSpec((1, HID), lambda i: (0, 0)),
            pl.BlockSpec((HID, D), lambda i: (0, 0)),
        ],
        out_specs=pl.BlockSpec((rb, D), lambda i: (i, 0)),
        out_shape=jax.ShapeDtypeStruct((NPAD, D), jnp.float32),
    )(acc1, dis, g1, W1, b1.reshape(1, HID), W2)

    acc2 = _spmm(g2, src_p, dst_p, zeros_nd)

    rf = 2000
    out = pl.pallas_call(
        _final_body,
        grid=(N // rf,),
        in_specs=[
            pl.BlockSpec((NSC, rf, D), lambda i: (0, i, 0)),
            pl.BlockSpec((rf, 1), lambda i: (i, 0)),
            pl.BlockSpec((rf, D), lambda i: (i, 0)),
            pl.BlockSpec((1, D), lambda i: (0, 0)),
        ],
        out_specs=pl.BlockSpec((rf, D), lambda i: (i, 0)),
        out_shape=jax.ShapeDtypeStruct((N, D), jnp.float32),
    )(acc2, dis, g2, b2.reshape(1, D))
    return out


# speedup vs baseline: 2.8327x; 1.0064x over previous
import functools

import jax
import jax.numpy as jnp
from jax import lax
from jax.experimental import pallas as pl
from jax.experimental.pallas import tpu as pltpu
from jax.experimental.pallas import tpu_sc as plsc

N = 10000
NPAD = 10240
E = 320000
D = 128
HID = 256
NSC = 2
NTILE = 16
NW = NSC * NTILE
KC = 128
NCK = 80
EPT = NCK * KC
EPAD = NW * EPT
RPT = NPAD // NTILE
NBUF = 4

_mesh = plsc.VectorSubcoreMesh(core_axis_name="c", subcore_axis_name="s")


@functools.partial(
    pl.kernel,
    out_type=jax.ShapeDtypeStruct((NSC, NPAD, D), jnp.float32),
    mesh=_mesh,
    scratch_types=[
        pltpu.VMEM((NCK, KC), jnp.int32),
        pltpu.VMEM((KC, D), jnp.float32),
        pltpu.VMEM_SHARED((NPAD, D), jnp.float32),
        pltpu.SemaphoreType.DMA,
        pltpu.SemaphoreType.DMA,
    ],
)
def _degree(dst_hbm, zeros_hbm, ones_hbm, hist_hbm,
            dsts, ones_v, acc_sh, s0, s1):
    ssem = (s0, s1)
    c = lax.axis_index("c")
    s = lax.axis_index("s")
    wid = c * NTILE + s
    pltpu.sync_copy(zeros_hbm.at[pl.ds(s * RPT, RPT)],
                    acc_sh.at[pl.ds(s * RPT, RPT)])
    pltpu.sync_copy(dst_hbm.at[wid], dsts)
    pltpu.sync_copy(ones_hbm, ones_v)
    plsc.subcore_barrier()

    def body(g, carry):
        for b in range(2):
            j = g * 2 + b
            @pl.when(g >= 1)
            def _():
                pltpu.make_async_copy(
                    ones_v, acc_sh.at[dsts.at[j - 2]], ssem[b]).wait()
            pltpu.async_copy(ones_v, acc_sh.at[dsts.at[j]], ssem[b], add=True)
        return carry

    lax.fori_loop(0, NCK // 2, body, 0)
    for b in range(2):
        pltpu.make_async_copy(
            ones_v, acc_sh.at[dsts.at[NCK - 2 + b]], ssem[b]).wait()
    plsc.subcore_barrier()
    pltpu.sync_copy(acc_sh.at[pl.ds(s * RPT, RPT)],
                    hist_hbm.at[c, pl.ds(s * RPT, RPT)])


@functools.partial(
    pl.kernel,
    out_type=jax.ShapeDtypeStruct((NSC, NPAD, D), jnp.float32),
    mesh=_mesh,
    scratch_types=[
        pltpu.VMEM((NCK, KC), jnp.int32),
        pltpu.VMEM((2, KC), jnp.int32),
        pltpu.VMEM((KC, D), jnp.float32),
        pltpu.VMEM((KC, D), jnp.float32),
        pltpu.VMEM_SHARED((NPAD, D), jnp.float32),
        pltpu.SemaphoreType.DMA,
        pltpu.SemaphoreType.DMA,
        pltpu.SemaphoreType.DMA,
        pltpu.SemaphoreType.DMA,
        pltpu.SemaphoreType.DMA,
        pltpu.SemaphoreType.DMA,
    ],
)
def _spmm(g_hbm, src_hbm, dst_hbm, zeros_hbm, acc_hbm,
          srcs, dst_t, r0, r1, acc_sh,
          g0, g1, s0, s1, d0, d1):
    rows = (r0, r1)
    gsem = (g0, g1)
    ssem = (s0, s1)
    dsem = (d0, d1)
    c = lax.axis_index("c")
    s = lax.axis_index("s")
    wid = c * NTILE + s
    pltpu.sync_copy(zeros_hbm.at[pl.ds(s * RPT, RPT)],
                    acc_sh.at[pl.ds(s * RPT, RPT)])
    pltpu.sync_copy(src_hbm.at[wid], srcs)
    plsc.subcore_barrier()

    def body(g, carry):
        for b in range(2):
            j = g * 2 + b

            @pl.when(j >= 2)
            def _():
                pltpu.make_async_copy(
                    rows[b], acc_sh.at[dst_t.at[b]], ssem[b]).wait()
            pltpu.async_copy(dst_hbm.at[wid, j], dst_t.at[b], dsem[b])
            pltpu.async_copy(g_hbm.at[srcs.at[j]], rows[b], gsem[b])
            pltpu.make_async_copy(
                g_hbm.at[srcs.at[j]], rows[b], gsem[b]).wait()
            pltpu.make_async_copy(
                dst_hbm.at[wid, j], dst_t.at[b], dsem[b]).wait()
            pltpu.async_copy(rows[b], acc_sh.at[dst_t.at[b]], ssem[b],
                             add=True)
        return carry

    lax.fori_loop(0, NCK // 2, body, 0)
    for b in range(2):
        pltpu.make_async_copy(
            rows[b], acc_sh.at[dst_t.at[b]], ssem[b]).wait()
    plsc.subcore_barrier()
    pltpu.sync_copy(acc_sh.at[pl.ds(s * RPT, RPT)],
                    acc_hbm.at[c, pl.ds(s * RPT, RPT)])


def _dis_g1_body(hist_ref, x_ref, dis_ref, g1_ref):
    deg = hist_ref[0, :, 0:1] + hist_ref[1, :, 0:1] + 1.0
    dis = lax.rsqrt(deg)
    dis_ref[...] = dis
    g1_ref[...] = x_ref[...] * dis


def _mid_body(acc_ref, dis_ref, g1_ref, w1_ref, b1_ref, w2_ref, g2_ref):
    dis = dis_ref[...]
    s1 = dis * (acc_ref[0] + acc_ref[1] + g1_ref[...])
    h = jnp.maximum(
        jnp.dot(s1, w1_ref[...], preferred_element_type=jnp.float32)
        + b1_ref[...], 0.0)
    t = jnp.dot(h, w2_ref[...], preferred_element_type=jnp.float32)
    g2_ref[...] = dis * t


def _final_body(acc_ref, dis_ref, g2_ref, b2_ref, out_ref):
    dis = dis_ref[...]
    out_ref[...] = dis * (acc_ref[0] + acc_ref[1] + g2_ref[...]) + b2_ref[...]


def kernel(x, edge_index, W1, b1, W2, b2):
    src = edge_index[0].astype(jnp.int32)
    dst = edge_index[1].astype(jnp.int32)
    npad_w = (EPAD - E) // NW
    nreal_w = E // NW
    pad_src = (jnp.arange(NW * npad_w, dtype=jnp.int32) * 37) % N
    pad_dst = N + jnp.arange(NW * npad_w, dtype=jnp.int32) % (NPAD - N)
    src_p = jnp.concatenate(
        [src.reshape(NW, nreal_w), pad_src.reshape(NW, npad_w)],
        axis=1).reshape(NW, NCK, KC)
    dst_p = jnp.concatenate(
        [dst.reshape(NW, nreal_w), pad_dst.reshape(NW, npad_w)],
        axis=1).reshape(NW, NCK, KC)
    x_pad = jnp.zeros((NPAD, D), jnp.float32).at[:N].set(x)
    zeros_nd = jnp.zeros((NPAD, D), jnp.float32)
    hist = _degree(dst_p, zeros_nd, jnp.ones((KC, D), jnp.float32))

    rb = 1024
    dis, g1 = pl.pallas_call(
        _dis_g1_body,
        grid=(NPAD // rb,),
        in_specs=[
            pl.BlockSpec((NSC, rb, D), lambda i: (0, i, 0)),
            pl.BlockSpec((rb, D), lambda i: (i, 0)),
        ],
        out_specs=[
            pl.BlockSpec((rb, 1), lambda i: (i, 0)),
            pl.BlockSpec((rb, D), lambda i: (i, 0)),
        ],
        out_shape=[
            jax.ShapeDtypeStruct((NPAD, 1), jnp.float32),
            jax.ShapeDtypeStruct((NPAD, D), jnp.float32),
        ],
    )(hist, x_pad)

    acc1 = _spmm(g1, src_p, dst_p, zeros_nd)

    g2 = pl.pallas_call(
        _mid_body,
        grid=(NPAD // rb,),
        in_specs=[
            pl.BlockSpec((NSC, rb, D), lambda i: (0, i, 0)),
            pl.BlockSpec((rb, 1), lambda i: (i, 0)),
            pl.BlockSpec((rb, D), lambda i: (i, 0)),
            pl.BlockSpec((D, HID), lambda i: (0, 0)),
            pl.BlockSpec((1, HID), lambda i: (0, 0)),
            pl.BlockSpec((HID, D), lambda i: (0, 0)),
        ],
        out_specs=pl.BlockSpec((rb, D), lambda i: (i, 0)),
        out_shape=jax.ShapeDtypeStruct((NPAD, D), jnp.float32),
    )(acc1, dis, g1, W1, b1.reshape(1, HID), W2)

    acc2 = _spmm(g2, src_p, dst_p, zeros_nd)

    rf = 2000
    out = pl.pallas_call(
        _final_body,
        grid=(N // rf,),
        in_specs=[
            pl.BlockSpec((NSC, rf, D), lambda i: (0, i, 0)),
            pl.BlockSpec((rf, 1), lambda i: (i, 0)),
            pl.BlockSpec((rf, D), lambda i: (i, 0)),
            pl.BlockSpec((1, D), lambda i: (0, 0)),
        ],
        out_specs=pl.BlockSpec((rf, D), lambda i: (i, 0)),
        out_shape=jax.ShapeDtypeStruct((N, D), jnp.float32),
    )(acc2, dis, g2, b2.reshape(1, D))
    return out


# speedup vs baseline: 2.8418x; 1.0032x over previous
import functools

import jax
import jax.numpy as jnp
from jax import lax
from jax.experimental import pallas as pl
from jax.experimental.pallas import tpu as pltpu
from jax.experimental.pallas import tpu_sc as plsc

N = 10000
NPAD = 10240
E = 320000
D = 128
HID = 256
NSC = 2
NTILE = 16
NW = NSC * NTILE
KC = 128
NCK = 80
EPT = NCK * KC
EPAD = NW * EPT
RPT = NPAD // NTILE
NBUF = 4

_mesh = plsc.VectorSubcoreMesh(core_axis_name="c", subcore_axis_name="s")


@functools.partial(
    pl.kernel,
    out_type=jax.ShapeDtypeStruct((NSC, NPAD, D), jnp.float32),
    mesh=_mesh,
    scratch_types=[
        pltpu.VMEM((NCK, KC), jnp.int32),
        pltpu.VMEM((KC, D), jnp.float32),
        pltpu.VMEM_SHARED((NPAD, D), jnp.float32),
        pltpu.SemaphoreType.DMA,
        pltpu.SemaphoreType.DMA,
    ],
)
def _degree(dst_hbm, zeros_hbm, ones_hbm, hist_hbm,
            dsts, ones_v, acc_sh, s0, s1):
    ssem = (s0, s1)
    c = lax.axis_index("c")
    s = lax.axis_index("s")
    wid = c * NTILE + s
    pltpu.sync_copy(zeros_hbm.at[pl.ds(s * RPT, RPT)],
                    acc_sh.at[pl.ds(s * RPT, RPT)])
    pltpu.sync_copy(dst_hbm.at[wid], dsts)
    pltpu.sync_copy(ones_hbm, ones_v)
    plsc.subcore_barrier()

    def body(g, carry):
        for b in range(2):
            j = g * 2 + b
            @pl.when(j >= 1)
            def _():
                pltpu.make_async_copy(
                    ones_v, acc_sh.at[dsts.at[j - 1]], ssem[1 - b]).wait()
            pltpu.async_copy(ones_v, acc_sh.at[dsts.at[j]], ssem[b], add=True)
        return carry

    lax.fori_loop(0, NCK // 2, body, 0)
    pltpu.make_async_copy(
        ones_v, acc_sh.at[dsts.at[NCK - 1]], ssem[1]).wait()
    plsc.subcore_barrier()
    pltpu.sync_copy(acc_sh.at[pl.ds(s * RPT, RPT)],
                    hist_hbm.at[c, pl.ds(s * RPT, RPT)])


@functools.partial(
    pl.kernel,
    out_type=jax.ShapeDtypeStruct((NSC, NPAD, D), jnp.float32),
    mesh=_mesh,
    scratch_types=[
        pltpu.VMEM((NCK, KC), jnp.int32),
        pltpu.VMEM((2, KC), jnp.int32),
        pltpu.VMEM((KC, D), jnp.float32),
        pltpu.VMEM((KC, D), jnp.float32),
        pltpu.VMEM_SHARED((NPAD, D), jnp.float32),
        pltpu.SemaphoreType.DMA,
        pltpu.SemaphoreType.DMA,
        pltpu.SemaphoreType.DMA,
        pltpu.SemaphoreType.DMA,
        pltpu.SemaphoreType.DMA,
        pltpu.SemaphoreType.DMA,
    ],
)
def _spmm(g_hbm, src_hbm, dst_hbm, zeros_hbm, acc_hbm,
          srcs, dst_t, r0, r1, acc_sh,
          g0, g1, s0, s1, d0, d1):
    rows = (r0, r1)
    gsem = (g0, g1)
    ssem = (s0, s1)
    dsem = (d0, d1)
    c = lax.axis_index("c")
    s = lax.axis_index("s")
    wid = c * NTILE + s
    pltpu.sync_copy(zeros_hbm.at[pl.ds(s * RPT, RPT)],
                    acc_sh.at[pl.ds(s * RPT, RPT)])
    pltpu.sync_copy(src_hbm.at[wid], srcs)
    pltpu.async_copy(dst_hbm.at[wid, 0], dst_t.at[0], dsem[0])
    pltpu.async_copy(g_hbm.at[srcs.at[0]], rows[0], gsem[0])
    plsc.subcore_barrier()

    def body(g, carry):
        for b in range(2):
            j = g * 2 + b
            pltpu.make_async_copy(
                g_hbm.at[srcs.at[j]], rows[b], gsem[b]).wait()
            pltpu.make_async_copy(
                dst_hbm.at[wid, j], dst_t.at[b], dsem[b]).wait()

            @pl.when(j >= 1)
            def _():
                pltpu.make_async_copy(
                    rows[1 - b], acc_sh.at[dst_t.at[1 - b]],
                    ssem[1 - b]).wait()
            pltpu.async_copy(rows[b], acc_sh.at[dst_t.at[b]], ssem[b],
                             add=True)

            @pl.when(j + 1 < NCK)
            def _():
                pltpu.async_copy(dst_hbm.at[wid, j + 1], dst_t.at[1 - b],
                                 dsem[1 - b])
                pltpu.async_copy(g_hbm.at[srcs.at[j + 1]], rows[1 - b],
                                 gsem[1 - b])
        return carry

    lax.fori_loop(0, NCK // 2, body, 0)
    pltpu.make_async_copy(
        rows[1], acc_sh.at[dst_t.at[1]], ssem[1]).wait()
    plsc.subcore_barrier()
    pltpu.sync_copy(acc_sh.at[pl.ds(s * RPT, RPT)],
                    acc_hbm.at[c, pl.ds(s * RPT, RPT)])


def _dis_g1_body(hist_ref, x_ref, dis_ref, g1_ref):
    deg = hist_ref[0, :, 0:1] + hist_ref[1, :, 0:1] + 1.0
    dis = lax.rsqrt(deg)
    dis_ref[...] = dis
    g1_ref[...] = x_ref[...] * dis


def _mid_body(acc_ref, dis_ref, g1_ref, w1_ref, b1_ref, w2_ref, g2_ref):
    dis = dis_ref[...]
    s1 = dis * (acc_ref[0] + acc_ref[1] + g1_ref[...])
    h = jnp.maximum(
        jnp.dot(s1, w1_ref[...], preferred_element_type=jnp.float32)
        + b1_ref[...], 0.0)
    t = jnp.dot(h, w2_ref[...], preferred_element_type=jnp.float32)
    g2_ref[...] = dis * t


def _final_body(acc_ref, dis_ref, g2_ref, b2_ref, out_ref):
    dis = dis_ref[...]
    out_ref[...] = dis * (acc_ref[0] + acc_ref[1] + g2_ref[...]) + b2_ref[...]


def kernel(x, edge_index, W1, b1, W2, b2):
    src = edge_index[0].astype(jnp.int32)
    dst = edge_index[1].astype(jnp.int32)
    npad_w = (EPAD - E) // NW
    nreal_w = E // NW
    pad_src = (jnp.arange(NW * npad_w, dtype=jnp.int32) * 37) % N
    pad_dst = N + jnp.arange(NW * npad_w, dtype=jnp.int32) % (NPAD - N)
    src_p = jnp.concatenate(
        [src.reshape(NW, nreal_w), pad_src.reshape(NW, npad_w)],
        axis=1).reshape(NW, NCK, KC)
    dst_p = jnp.concatenate(
        [dst.reshape(NW, nreal_w), pad_dst.reshape(NW, npad_w)],
        axis=1).reshape(NW, NCK, KC)
    x_pad = jnp.zeros((NPAD, D), jnp.float32).at[:N].set(x)
    zeros_nd = jnp.zeros((NPAD, D), jnp.float32)
    hist = _degree(dst_p, zeros_nd, jnp.ones((KC, D), jnp.float32))

    rb = 1024
    dis, g1 = pl.pallas_call(
        _dis_g1_body,
        grid=(NPAD // rb,),
        in_specs=[
            pl.BlockSpec((NSC, rb, D), lambda i: (0, i, 0)),
            pl.BlockSpec((rb, D), lambda i: (i, 0)),
        ],
        out_specs=[
            pl.BlockSpec((rb, 1), lambda i: (i, 0)),
            pl.BlockSpec((rb, D), lambda i: (i, 0)),
        ],
        out_shape=[
            jax.ShapeDtypeStruct((NPAD, 1), jnp.float32),
            jax.ShapeDtypeStruct((NPAD, D), jnp.float32),
        ],
    )(hist, x_pad)

    acc1 = _spmm(g1, src_p, dst_p, zeros_nd)

    g2 = pl.pallas_call(
        _mid_body,
        grid=(NPAD // rb,),
        in_specs=[
            pl.BlockSpec((NSC, rb, D), lambda i: (0, i, 0)),
            pl.BlockSpec((rb, 1), lambda i: (i, 0)),
            pl.BlockSpec((rb, D), lambda i: (i, 0)),
            pl.BlockSpec((D, HID), lambda i: (0, 0)),
            pl.BlockSpec((1, HID), lambda i: (0, 0)),
            pl.BlockSpec((HID, D), lambda i: (0, 0)),
        ],
        out_specs=pl.BlockSpec((rb, D), lambda i: (i, 0)),
        out_shape=jax.ShapeDtypeStruct((NPAD, D), jnp.float32),
    )(acc1, dis, g1, W1, b1.reshape(1, HID), W2)

    acc2 = _spmm(g2, src_p, dst_p, zeros_nd)

    rf = 2000
    out = pl.pallas_call(
        _final_body,
        grid=(N // rf,),
        in_specs=[
            pl.BlockSpec((NSC, rf, D), lambda i: (0, i, 0)),
            pl.BlockSpec((rf, 1), lambda i: (i, 0)),
            pl.BlockSpec((rf, D), lambda i: (i, 0)),
            pl.BlockSpec((1, D), lambda i: (0, 0)),
        ],
        out_specs=pl.BlockSpec((rf, D), lambda i: (i, 0)),
        out_shape=jax.ShapeDtypeStruct((N, D), jnp.float32),
    )(acc2, dis, g2, b2.reshape(1, D))
    return out
